# baseline (speedup 1.0000x reference)
import jax
import jax.numpy as jnp
from jax import lax
from jax.experimental import pallas as pl
from jax.experimental.pallas import tpu as pltpu
from jax.experimental.pallas import tpu_sc as plsc

_VOCAB = 100000
_M = 10
_K = 256
_SUB = 30
_NTOK = 4096 * 50
_NW = 32
_TPW = _NTOK // _NW
_SPB = 8
_BLK = _SPB * 50
_NBLK = _TPW // _BLK
_CBW = _M * _K * _SUB
_D = _M * _SUB
_LINES = _VOCAB // 32


def _sc_body(ids_hbm, vec_hbm, cw_hbm, out_hbm,
             cb_v, ids0, ids1, ring0, ring1, out_v,
             si0, si1, sg0, sg1, so0, so1):
    cid = lax.axis_index("c")
    sid = lax.axis_index("s")
    wid = sid * 2 + cid
    tok0 = wid * _TPW
    sent0 = wid * (_TPW // 50)
    lane = lax.iota(jnp.int32, 16)
    idsb, sib = (ids0, ids1), (si0, si1)
    ringb, sgb = (ring0, ring1), (sg0, sg1)

    pltpu.sync_copy(cw_hbm, cb_v)
    pltpu.async_copy(ids_hbm.at[pl.ds(tok0, _BLK)], ids0, si0)

    def issue_gather(ids_ref, base, act, rp):
        idv = plsc.load_gather(ids_ref, [base + lane], mask=lane < act)
        pltpu.async_copy(vec_hbm.at[idv >> 5], ringb[rp], sgb[rp])

    def group(blk, pb, kp, j, ids_ref):
        k = kp * 2 + j
        kg = pb * 8 + k
        act = jnp.where((k & 3) == 3, 2, 16)
        m = lane < act
        base = pb * 100 + (k >> 2) * 50 + (k & 3) * 16
        slot = kp >> 1
        s_loc = blk * _SPB + pb * 2 + (k >> 2)

        pltpu.make_async_copy(vec_hbm.at[lane], ringb[j], sgb[j]).wait()

        @pl.when(kg < 31)
        def _():
            nk = k + 1
            nbase = pb * 100 + (nk >> 2) * 50 + (nk & 3) * 16
            nact = jnp.where((nk & 3) == 3, 2, 16)
            issue_gather(ids_ref, nbase, nact, 1 - j)

        if j == 0:
            @pl.when((kp == 0) & (s_loc >= 2))
            def _():
                pltpu.make_async_copy(
                    out_v.at[pl.ds(0, 1)],
                    out_hbm.at[pl.ds(sent0, 1)], so0).wait()

            @pl.when((kp == 2) & (s_loc >= 2))
            def _():
                pltpu.make_async_copy(
                    out_v.at[pl.ds(1, 1)],
                    out_hbm.at[pl.ds(sent0, 1)], so1).wait()

        idv = plsc.load_gather(ids_ref, [base + lane], mask=m)
        colw = (idv & 31) * 4
        lvec = (k & 3) * 16 + lane
        slotv = jnp.full((16,), slot, jnp.int32)
        for i in range(_M):
            w = plsc.load_gather(ringb[j], [lane, colw + (i >> 2)])
            c16 = (w >> ((i & 3) * 8)) & 255
            bvec = c16 * _SUB + i * (_K * _SUB)
            col0 = jnp.full((16,), i * _SUB, jnp.int32)

            @plsc.parallel_loop(0, _SUB, unroll=6)
            def _dl(d):
                vals = plsc.load_gather(cb_v, [bvec + d])
                plsc.store_scatter(out_v, [slotv, lvec, col0 + d], vals,
                                   mask=m)

        if j == 1:
            @pl.when(kp == 1)
            def _():
                pltpu.async_copy(out_v.at[pl.ds(0, 1)],
                                 out_hbm.at[pl.ds(sent0 + s_loc, 1)], so0)

            @pl.when(kp == 3)
            def _():
                pltpu.async_copy(out_v.at[pl.ds(1, 1)],
                                 out_hbm.at[pl.ds(sent0 + s_loc, 1)], so1)

    def block(blk, bi):
        pltpu.make_async_copy(
            ids_hbm.at[pl.ds(tok0, _BLK)], idsb[bi], sib[bi]).wait()

        @pl.when(blk + 1 < _NBLK)
        def _():
            pltpu.async_copy(
                ids_hbm.at[pl.ds(tok0 + (blk + 1) * _BLK, _BLK)],
                idsb[1 - bi], sib[1 - bi])

        issue_gather(idsb[bi], 0, 16, 0)

        def q_body(q, carry):
            pb = q >> 2
            kp = q & 3
            group(blk, pb, kp, 0, idsb[bi])
            group(blk, pb, kp, 1, idsb[bi])
            return carry

        lax.fori_loop(0, 16, q_body, 0)

    def bp_body(bp, carry):
        block(bp * 2, 0)
        block(bp * 2 + 1, 1)
        return carry

    lax.fori_loop(0, _NBLK // 2, bp_body, 0)

    pltpu.make_async_copy(out_v.at[pl.ds(0, 1)],
                          out_hbm.at[pl.ds(sent0, 1)], so0).wait()
    pltpu.make_async_copy(out_v.at[pl.ds(1, 1)],
                          out_hbm.at[pl.ds(sent0, 1)], so1).wait()


def kernel(input_ids, codewords, vectors):
    ids = input_ids.reshape(_NTOK)
    cw = codewords.reshape(_CBW)
    vb = jnp.pad(vectors.astype(jnp.uint8), ((0, 0), (0, 6)))
    vec_lines = lax.bitcast_convert_type(
        vb.reshape(_VOCAB, 4, 4), jnp.int32).reshape(_LINES, 128)
    mesh = plsc.VectorSubcoreMesh(core_axis_name="c", subcore_axis_name="s")
    out = pl.kernel(
        _sc_body,
        out_type=jax.ShapeDtypeStruct((4096, 50, _D), jnp.float32),
        mesh=mesh,
        compiler_params=pltpu.CompilerParams(
            use_tc_tiling_on_sc=True, needs_layout_passes=False),
        scratch_types=[
            pltpu.VMEM((_CBW,), jnp.float32),
            pltpu.VMEM((_BLK,), jnp.int32),
            pltpu.VMEM((_BLK,), jnp.int32),
            pltpu.VMEM((16, 128), jnp.int32),
            pltpu.VMEM((16, 128), jnp.int32),
            pltpu.VMEM((2, 50, _D), jnp.float32),
            pltpu.SemaphoreType.DMA,
            pltpu.SemaphoreType.DMA,
            pltpu.SemaphoreType.DMA,
            pltpu.SemaphoreType.DMA,
            pltpu.SemaphoreType.DMA,
            pltpu.SemaphoreType.DMA,
        ],
    )(ids, vec_lines, cw)
    return out
